# baseline (speedup 1.0000x reference)
import functools
import math

import jax
import jax.numpy as jnp
from jax import lax
from jax.experimental import pallas as pl
from jax.experimental.pallas import tpu as pltpu
from jax.experimental.pallas import tpu_sc as plsc

_NCG = 4
_T = 128
_ABLATE = ""
_PAD = 1


def _emb_body(dims, scale, gtab_hbm, phon_hbm, a1_hbm, f2_hbm, out_hbm,
              gt_ref, ip0, ip1, if0, if1, ia0, ia1, ob0, ob1,
              is0, is1, os0, os1):
    BL, V, C, crange = dims
    ipb, ifb, iab = [ip0, ip1], [if0, if1], [ia0, ia1]
    obb, isem, osem = [ob0, ob1], [is0, is1], [os0, os1]
    info = plsc.get_sparse_core_info()
    nw = info.num_cores * info.num_subcores
    ntg = nw // _NCG
    wid = lax.axis_index("s") * info.num_cores + lax.axis_index("c")
    cg = wid % _NCG
    tg = wid // _NCG
    span = BL // ntg
    nchunks = span // _T
    tile_t0 = tg * span
    c0ch = cg * crange

    pltpu.sync_copy(gtab_hbm.at[pl.ds(c0ch, crange), :], gt_ref)

    def fire_in(ci, par):
        t0 = tile_t0 + ci * _T
        pltpu.async_copy(phon_hbm.at[pl.ds(t0, _T)], ipb[par], isem[par])
        pltpu.async_copy(f2_hbm.at[pl.ds(t0, _T)], ifb[par], isem[par])
        pltpu.async_copy(a1_hbm.at[pl.ds(t0, _T)], iab[par], isem[par])

    def drain_in(par):
        pltpu.make_async_copy(phon_hbm.at[pl.ds(0, _T)], ipb[par], isem[par]).wait()
        pltpu.make_async_copy(f2_hbm.at[pl.ds(0, _T)], ifb[par], isem[par]).wait()
        pltpu.make_async_copy(a1_hbm.at[pl.ds(0, _T)], iab[par], isem[par]).wait()

    def drain_out(p):
        pltpu.make_async_copy(out_hbm.at[pl.ds(0, _T), pl.ds(0, crange)],
                              obb[p].at[:, pl.ds(0, crange)], osem[p]).wait()

    def fire_out(ci, p):
        t0 = tile_t0 + ci * _T
        pltpu.async_copy(obb[p].at[:, pl.ds(0, crange)],
                         out_hbm.at[pl.ds(t0, _T), pl.ds(c0ch, crange)],
                         osem[p])

    def compute(par, p):
        ipr, ifr, iar, ob = ipb[par], ifb[par], iab[par], obb[p]

        def kbody(k, kc):
            tk = k * 16
            ipv = ipr[pl.ds(tk, 16)]
            ifv = ifr[pl.ds(tk, 16)]
            av = iar[pl.ds(tk, 16)]
            tokv = lax.iota(jnp.int32, 16) + tk
            for c in range(crange):
                ch = c0ch + c
                cvec = jnp.full((16,), c, jnp.int32)
                iv = jnp.where(ch < C, ipv, ifv)
                g = plsc.load_gather(gt_ref, [cvec, iv])
                vf = jnp.where(ch < 2 * C, g * scale, av)
                plsc.store_scatter(ob, [tokv, cvec], vf)
            return kc

        lax.fori_loop(0, _T // 16, kbody, 0)

    fire_in(0, 0)
    fire_in(1, 1)

    def hbody(h, hc):
        ci0 = 2 * h
        drain_in(0)

        @pl.when(h > 0)
        def _():
            drain_out(0)

        if _ABLATE != "dma_only":
            compute(0, 0)
        fire_out(ci0, 0)

        @pl.when(ci0 + 2 < nchunks)
        def _():
            fire_in(ci0 + 2, 0)

        drain_in(1)

        @pl.when(h > 0)
        def _():
            drain_out(1)

        if _ABLATE != "dma_only":
            compute(1, 1)
        fire_out(ci0 + 1, 1)

        @pl.when(ci0 + 3 < nchunks)
        def _():
            fire_in(ci0 + 3, 1)

        return hc

    lax.fori_loop(0, nchunks // 2, hbody, 0)
    drain_out(0)
    drain_out(1)


def kernel(phoneme, a1, f2, phoneme_table, f2_table):
    B, L = phoneme.shape
    V, C = phoneme_table.shape
    BL = B * L
    scale = math.sqrt(C)
    info = plsc.get_sparse_core_info()
    nw = info.num_cores * info.num_subcores
    ntg = nw // _NCG
    crange = 3 * C // _NCG
    assert (3 * C) % _NCG == 0 and BL % (ntg * 2 * _T) == 0 and _T % 16 == 0

    gtab = jnp.concatenate(
        [jnp.transpose(phoneme_table), jnp.transpose(f2_table),
         jnp.zeros((crange * _NCG - 2 * C, V), jnp.float32)], axis=0)
    phoneme = phoneme.astype(jnp.int32).reshape(-1)
    f2 = f2.astype(jnp.int32).reshape(-1)
    a1 = a1.astype(jnp.float32).reshape(-1)

    mesh = plsc.VectorSubcoreMesh(core_axis_name="c", subcore_axis_name="s")
    run = pl.kernel(
        functools.partial(_emb_body, (BL, V, C, crange), scale),
        out_type=jax.ShapeDtypeStruct((BL, 3 * C), jnp.float32),
        mesh=mesh,
        compiler_params=pltpu.CompilerParams(
            needs_layout_passes=False, use_tc_tiling_on_sc=False),
        scratch_types=[
            pltpu.VMEM((crange, V), jnp.float32),
            pltpu.VMEM((_T,), jnp.int32),
            pltpu.VMEM((_T,), jnp.int32),
            pltpu.VMEM((_T,), jnp.int32),
            pltpu.VMEM((_T,), jnp.int32),
            pltpu.VMEM((_T,), jnp.float32),
            pltpu.VMEM((_T,), jnp.float32),
            pltpu.VMEM((_T, 3 * C // _NCG + _PAD), jnp.float32),
            pltpu.VMEM((_T, 3 * C // _NCG + _PAD), jnp.float32),
            pltpu.SemaphoreType.DMA,
            pltpu.SemaphoreType.DMA,
            pltpu.SemaphoreType.DMA,
            pltpu.SemaphoreType.DMA,
        ],
    )
    out = run(gtab, phoneme, a1, f2)
    return jnp.swapaxes(out.reshape(B, L, 3 * C), -1, -2)


# speedup vs baseline: 6.3699x; 6.3699x over previous
import functools
import math

import jax
import jax.numpy as jnp
from jax import lax
from jax.experimental import pallas as pl
from jax.experimental.pallas import tpu as pltpu
from jax.experimental.pallas import tpu_sc as plsc

_T = 128


def _emb_body(dims, tabP_hbm, tabF_hbm, phon_hbm, a1_hbm, f2_hbm, out_hbm,
              ip0, ip1, if0, if1, ia0, ia1, ob0, ob1,
              is0, is1, gs0, gs1, os0, os1):
    BL, V, C = dims
    ipb, ifb, iab = [ip0, ip1], [if0, if1], [ia0, ia1]
    obb = [ob0, ob1]
    isem, gsem, osem = [is0, is1], [gs0, gs1], [os0, os1]
    info = plsc.get_sparse_core_info()
    nw = info.num_cores * info.num_subcores
    wid = lax.axis_index("s") * info.num_cores + lax.axis_index("c")
    span = BL // nw
    nchunks = span // _T
    tile_t0 = wid * span

    def fire_in(ci, par):
        t0 = tile_t0 + ci * _T
        pltpu.async_copy(phon_hbm.at[pl.ds(t0, _T)], ipb[par], isem[par])
        pltpu.async_copy(f2_hbm.at[pl.ds(t0, _T)], ifb[par], isem[par])
        pltpu.async_copy(a1_hbm.at[pl.ds(t0, _T)], iab[par], isem[par])

    def drain_in(par):
        pltpu.make_async_copy(phon_hbm.at[pl.ds(0, _T)], ipb[par], isem[par]).wait()
        pltpu.make_async_copy(f2_hbm.at[pl.ds(0, _T)], ifb[par], isem[par]).wait()
        pltpu.make_async_copy(a1_hbm.at[pl.ds(0, _T)], iab[par], isem[par]).wait()

    def fire_gath(p):
        pltpu.async_copy(tabP_hbm.at[ipb[p]], obb[p].at[:, pl.ds(0, C)], gsem[p])
        pltpu.async_copy(tabF_hbm.at[ifb[p]], obb[p].at[:, pl.ds(C, C)], gsem[p])

    def drain_gath(p):
        pltpu.make_async_copy(tabP_hbm.at[ipb[p]], obb[p].at[:, pl.ds(0, C)],
                              gsem[p]).wait()
        pltpu.make_async_copy(tabF_hbm.at[ifb[p]], obb[p].at[:, pl.ds(C, C)],
                              gsem[p]).wait()

    def a1_fill(p):
        iar, ob = iab[p], obb[p]

        def kbody(k, kc):
            tk = k * 16
            av = iar[pl.ds(tk, 16)]
            for j in range(16):
                val = jnp.full((16,), av[j], jnp.float32)
                for i in range(C // 16):
                    ob[tk + j, pl.ds(2 * C + 16 * i, 16)] = val
            return kc

        lax.fori_loop(0, _T // 16, kbody, 0)

    def drain_out(p):
        pltpu.make_async_copy(out_hbm.at[pl.ds(0, _T), :], obb[p], osem[p]).wait()

    def fire_out(ci, p):
        t0 = tile_t0 + ci * _T
        pltpu.async_copy(obb[p], out_hbm.at[pl.ds(t0, _T), :], osem[p])

    fire_in(0, 0)
    fire_in(1, 1)

    def hbody(h, hc):
        ci0 = 2 * h
        for p in (0, 1):
            ci = ci0 + p
            drain_in(p)

            @pl.when(h > 0)
            def _():
                drain_out(p)

            fire_gath(p)
            a1_fill(p)
            drain_gath(p)

            @pl.when(ci + 2 < nchunks)
            def _():
                fire_in(ci + 2, p)

            fire_out(ci, p)
        return hc

    lax.fori_loop(0, nchunks // 2, hbody, 0)
    drain_out(0)
    drain_out(1)


def kernel(phoneme, a1, f2, phoneme_table, f2_table):
    B, L = phoneme.shape
    V, C = phoneme_table.shape
    BL = B * L
    scale = math.sqrt(C)
    info = plsc.get_sparse_core_info()
    nw = info.num_cores * info.num_subcores
    assert BL % (nw * 2 * _T) == 0 and C % 16 == 0

    tabP = phoneme_table * scale
    tabF = f2_table * scale
    phoneme = phoneme.astype(jnp.int32).reshape(-1)
    f2 = f2.astype(jnp.int32).reshape(-1)
    a1 = a1.astype(jnp.float32).reshape(-1)

    mesh = plsc.VectorSubcoreMesh(core_axis_name="c", subcore_axis_name="s")
    run = pl.kernel(
        functools.partial(_emb_body, (BL, V, C)),
        out_type=jax.ShapeDtypeStruct((BL, 3 * C), jnp.float32),
        mesh=mesh,
        compiler_params=pltpu.CompilerParams(needs_layout_passes=False),
        scratch_types=[
            pltpu.VMEM((_T,), jnp.int32),
            pltpu.VMEM((_T,), jnp.int32),
            pltpu.VMEM((_T,), jnp.int32),
            pltpu.VMEM((_T,), jnp.int32),
            pltpu.VMEM((_T,), jnp.float32),
            pltpu.VMEM((_T,), jnp.float32),
            pltpu.VMEM((_T, 384), jnp.float32),
            pltpu.VMEM((_T, 384), jnp.float32),
            pltpu.SemaphoreType.DMA,
            pltpu.SemaphoreType.DMA,
            pltpu.SemaphoreType.DMA,
            pltpu.SemaphoreType.DMA,
            pltpu.SemaphoreType.DMA,
            pltpu.SemaphoreType.DMA,
        ],
    )
    out = run(tabP, tabF, phoneme, a1, f2)
    return jnp.swapaxes(out.reshape(B, L, 3 * C), -1, -2)


# speedup vs baseline: 6.3769x; 1.0011x over previous
import functools
import math

import jax
import jax.numpy as jnp
from jax import lax
from jax.experimental import pallas as pl
from jax.experimental.pallas import tpu as pltpu
from jax.experimental.pallas import tpu_sc as plsc

_T = 64
_GRP = 4


def _emb_body(dims, tabP_hbm, tabF_hbm, phon_hbm, a1_hbm, f2_hbm, out_hbm,
              ip0, ip1, if0, if1, ia0, ia1, ob0, ob1, ob2, ob3,
              is0, is1, gs0, gs1, gs2, gs3, os0, os1, os2, os3):
    BL, V, C = dims
    ipb, ifb, iab = [ip0, ip1], [if0, if1], [ia0, ia1]
    obb = [ob0, ob1, ob2, ob3]
    isem = [is0, is1]
    gsem = [gs0, gs1, gs2, gs3]
    osem = [os0, os1, os2, os3]
    info = plsc.get_sparse_core_info()
    nw = info.num_cores * info.num_subcores
    wid = lax.axis_index("s") * info.num_cores + lax.axis_index("c")
    span = BL // nw
    nchunks = span // _T
    ngroups = nchunks // _GRP
    tile_t0 = wid * span
    tile_r0 = tile_t0 // _T

    def fire_in(g, par):
        t0 = tile_t0 + g * _GRP * _T
        pltpu.async_copy(phon_hbm.at[pl.ds(t0, _GRP * _T)], ipb[par], isem[par])
        pltpu.async_copy(f2_hbm.at[pl.ds(t0, _GRP * _T)], ifb[par], isem[par])
        pltpu.async_copy(a1_hbm.at[pl.ds(t0, _GRP * _T)], iab[par], isem[par])

    def drain_in(par):
        pltpu.make_async_copy(phon_hbm.at[pl.ds(0, _GRP * _T)], ipb[par], isem[par]).wait()
        pltpu.make_async_copy(f2_hbm.at[pl.ds(0, _GRP * _T)], ifb[par], isem[par]).wait()
        pltpu.make_async_copy(a1_hbm.at[pl.ds(0, _GRP * _T)], iab[par], isem[par]).wait()

    def fire_gath(par, row, b):
        pltpu.async_copy(tabP_hbm.at[ipb[par].at[pl.ds(row * _T, _T)]],
                         obb[b].at[:, pl.ds(0, C)], gsem[b])
        pltpu.async_copy(tabF_hbm.at[ifb[par].at[pl.ds(row * _T, _T)]],
                         obb[b].at[:, pl.ds(C, C)], gsem[b])

    def drain_gath(par, row, b):
        pltpu.make_async_copy(tabP_hbm.at[ipb[par].at[pl.ds(row * _T, _T)]],
                              obb[b].at[:, pl.ds(0, C)], gsem[b]).wait()
        pltpu.make_async_copy(tabF_hbm.at[ifb[par].at[pl.ds(row * _T, _T)]],
                              obb[b].at[:, pl.ds(C, C)], gsem[b]).wait()

    def a1_fill(par, row, b):
        iar, ob = iab[par], obb[b]

        def kbody(k, kc):
            tk = k * 16
            av = iar[pl.ds(row * _T + tk, 16)]
            for j in range(16):
                val = jnp.full((16,), av[j], jnp.float32)
                for i in range(C // 16):
                    ob[tk + j, pl.ds(2 * C + 16 * i, 16)] = val
            return kc

        lax.fori_loop(0, _T // 16, kbody, 0)

    def drain_out(b):
        pltpu.make_async_copy(out_hbm.at[pl.ds(0, _T), :], obb[b], osem[b]).wait()

    def fire_out(ci, b):
        t0 = tile_t0 + ci * _T
        pltpu.async_copy(obb[b], out_hbm.at[pl.ds(t0, _T), :], osem[b])

    def do_group(g, ipar, afirst, hg):
        for j in range(_GRP):
            ci = g * _GRP + j
            tb = (j + 2) % 4

            @pl.when(ci + 2 < nchunks)
            def _(j=j, tb=tb, ipar=ipar):
                if j == 2:
                    drain_in(1 - ipar)
                if afirst and j < 2:
                    @pl.when(hg > 0)
                    def _(tb=tb):
                        drain_out(tb)
                else:
                    drain_out(tb)
                fire_gath(ipar if j < 2 else 1 - ipar, (j + 2) % 4, tb)

            a1_fill(ipar, j, j)
            drain_gath(ipar, j, j)
            if j == _GRP - 1:
                @pl.when(g + 2 < ngroups)
                def _(ipar=ipar):
                    fire_in(g + 2, ipar)
            fire_out(ci, j)

    fire_in(0, 0)
    fire_in(1, 1)
    drain_in(0)
    fire_gath(0, 0, 0)
    fire_gath(0, 1, 1)

    def hbody(hg, hc):
        do_group(2 * hg, 0, True, hg)
        do_group(2 * hg + 1, 1, False, hg)
        return hc

    lax.fori_loop(0, ngroups // 2, hbody, 0)
    drain_out(0)
    drain_out(1)
    drain_out(2)
    drain_out(3)


def kernel(phoneme, a1, f2, phoneme_table, f2_table):
    B, L = phoneme.shape
    V, C = phoneme_table.shape
    BL = B * L
    scale = math.sqrt(C)
    info = plsc.get_sparse_core_info()
    nw = info.num_cores * info.num_subcores
    assert BL % (nw * 2 * _GRP * _T) == 0 and C % 16 == 0 and _GRP == 4

    tabP = phoneme_table * scale
    tabF = f2_table * scale
    phoneme = phoneme.astype(jnp.int32).reshape(-1)
    f2 = f2.astype(jnp.int32).reshape(-1)
    a1 = a1.astype(jnp.float32).reshape(-1)

    mesh = plsc.VectorSubcoreMesh(core_axis_name="c", subcore_axis_name="s")
    run = pl.kernel(
        functools.partial(_emb_body, (BL, V, C)),
        out_type=jax.ShapeDtypeStruct((BL, 3 * C), jnp.float32),
        mesh=mesh,
        compiler_params=pltpu.CompilerParams(needs_layout_passes=False),
        scratch_types=(
            [pltpu.VMEM((_GRP * _T,), jnp.int32) for _ in range(2)] +
            [pltpu.VMEM((_GRP * _T,), jnp.int32) for _ in range(2)] +
            [pltpu.VMEM((_GRP * _T,), jnp.float32) for _ in range(2)] +
            [pltpu.VMEM((_T, 384), jnp.float32) for _ in range(4)] +
            [pltpu.SemaphoreType.DMA for _ in range(10)]
        ),
    )
    out = run(tabP, tabF, phoneme, a1, f2)
    return jnp.swapaxes(out.reshape(B, L, 3 * C), -1, -2)
